# baseline (speedup 1.0000x reference)
import functools
import math

import jax
import jax.numpy as jnp
from jax import lax
from jax.experimental import pallas as pl
from jax.experimental.pallas import tpu as pltpu
from jax.experimental.pallas import tpu_sc as plsc

VOCAB = 30522
H = 768
S = 512
B = 64
NC = 2
NS = 16
NW = NC * NS
SPW = S // NW
CH = 32
NCH = SPW * (B // CH)
NBUF = 4
BS = 32
EPS = 1e-12
SQRT_H = math.sqrt(float(H))


@functools.partial(
    pl.kernel,
    out_type=jax.ShapeDtypeStruct((S, B, H), jnp.float32),
    mesh=plsc.VectorSubcoreMesh(core_axis_name="c", subcore_axis_name="s"),
    scratch_types=[
        pltpu.VMEM((SPW, B), jnp.int32),
        pltpu.VMEM((B, 128), jnp.int32),
        pltpu.VMEM((CH, H), jnp.float32),
        pltpu.VMEM((CH, H), jnp.float32),
        pltpu.VMEM((CH, H), jnp.float32),
        pltpu.VMEM((CH, H), jnp.float32),
        pltpu.SemaphoreType.DMA,
        pltpu.SemaphoreType.DMA,
        pltpu.SemaphoreType.DMA,
        pltpu.SemaphoreType.DMA,
        pltpu.SemaphoreType.DMA,
        pltpu.SemaphoreType.DMA,
        pltpu.SemaphoreType.DMA,
        pltpu.SemaphoreType.DMA,
    ],
    compiler_params=pltpu.CompilerParams(needs_layout_passes=False),
)
def _gather_kernel(x, word, out, idx_v, idx2_v, b0, b1, b2, b3,
                   sg0, sg1, sg2, sg3, ss0, ss1, ss2, ss3):
    w = lax.axis_index("s") * NC + lax.axis_index("c")
    s0 = w * SPW

    s0_al = lax.div(w, 8) * 128
    s_off = lax.rem(w, 8) * SPW
    pltpu.sync_copy(x.at[:, pl.ds(s0_al, 128)], idx2_v)
    lanes = lax.iota(jnp.int32, 16)
    for s_l in range(SPW):
        col = jnp.full((16,), s_off + s_l, jnp.int32)
        for jb in range(B // 16):
            v = plsc.load_gather(idx2_v, [lanes + jb * 16, col])
            idx_v[s_l, jb * 16:(jb + 1) * 16] = v

    bufs = (b0, b1, b2, b3)
    gsems = (sg0, sg1, sg2, sg3)
    ssems = (ss0, ss1, ss2, ss3)

    def _idx_ref(c):
        return idx_v.at[lax.div(c, B // CH), pl.ds(lax.rem(c, B // CH) * CH, CH)]

    def _out_ref(c):
        return out.at[s0 + lax.div(c, B // CH),
                      pl.ds(lax.rem(c, B // CH) * CH, CH)]

    pltpu.async_copy(word.at[_idx_ref(0)], b0, sg0)
    pltpu.async_copy(word.at[_idx_ref(1)], b1, sg1)

    def _giter(g, _):
        for par in range(NBUF):
            c = g * NBUF + par
            buf = bufs[par]
            nxt = (par + 2) % NBUF

            @pl.when(c + 2 < NCH)
            def _():
                @pl.when(c >= 2)
                def _():
                    pltpu.make_async_copy(bufs[nxt], _out_ref(0),
                                          ssems[nxt]).wait()

                pltpu.async_copy(word.at[_idx_ref(c + 2)], bufs[nxt],
                                 gsems[nxt])

            pltpu.make_async_copy(word.at[pl.ds(0, CH)], buf,
                                  gsems[par]).wait()
            pltpu.async_copy(buf, _out_ref(c), ssems[par])
        return 0

    lax.fori_loop(0, NCH // NBUF, _giter, 0)
    for p in range(NBUF):
        pltpu.make_async_copy(bufs[p], _out_ref(0), ssems[p]).wait()


def _ln_body(scr, pos, typ, gamma, beta, out):
    e = scr[...] + pos[...][:, None, :] + typ[...][0][None, None, :]
    sum1 = jnp.sum(e, axis=-1, keepdims=True)
    sum2 = jnp.sum(e * e, axis=-1, keepdims=True)
    mean = sum1 * (1.0 / H)
    var = sum2 * (1.0 / H) - mean * mean
    a = lax.rsqrt(var + EPS)
    g = gamma[...][0] * SQRT_H
    b = beta[...][0] * SQRT_H
    out[...] = (e * a - mean * a) * g + b


_ln_kernel = pl.pallas_call(
    _ln_body,
    grid=(S // BS,),
    in_specs=[
        pl.BlockSpec((BS, B, H), lambda i: (i, 0, 0)),
        pl.BlockSpec((BS, H), lambda i: (i, 0)),
        pl.BlockSpec((2, H), lambda i: (0, 0)),
        pl.BlockSpec((1, H), lambda i: (0, 0)),
        pl.BlockSpec((1, H), lambda i: (0, 0)),
    ],
    out_specs=pl.BlockSpec((BS, B, H), lambda i: (i, 0, 0)),
    out_shape=jax.ShapeDtypeStruct((S, B, H), jnp.float32),
    compiler_params=pltpu.CompilerParams(
        dimension_semantics=("arbitrary",),
    ),
)


def kernel(x, word_emb, pos_emb, type_emb, ln_gamma, ln_beta):
    gathered = _gather_kernel(x, word_emb)
    return _ln_kernel(gathered, pos_emb, type_emb,
                      ln_gamma.reshape(1, H), ln_beta.reshape(1, H))


# speedup vs baseline: 1.0197x; 1.0197x over previous
import functools
import math

import jax
import jax.numpy as jnp
from jax import lax
from jax.experimental import pallas as pl
from jax.experimental.pallas import tpu as pltpu
from jax.experimental.pallas import tpu_sc as plsc

VOCAB = 30522
H = 768
S = 512
B = 64
NC = 2
NS = 16
NW = NC * NS
SPW = S // NW
CH = 16
NCH = SPW * (B // CH)
NBUF = 8
DEPTH = 4
BS = 32
EPS = 1e-12
SQRT_H = math.sqrt(float(H))


@functools.partial(
    pl.kernel,
    out_type=jax.ShapeDtypeStruct((S, B, H), jnp.float32),
    mesh=plsc.VectorSubcoreMesh(core_axis_name="c", subcore_axis_name="s"),
    scratch_types=(
        [pltpu.VMEM((SPW, B), jnp.int32)]
        + [pltpu.VMEM((CH, H), jnp.float32) for _ in range(NBUF)]
        + [pltpu.SemaphoreType.DMA for _ in range(2 * NBUF)]
    ),
    compiler_params=pltpu.CompilerParams(needs_layout_passes=False),
)
def _gather_kernel(xt, word, out, idx_v, *rest):
    bufs = rest[:NBUF]
    gsems = rest[NBUF:2 * NBUF]
    ssems = rest[2 * NBUF:]
    w = lax.axis_index("s") * NC + lax.axis_index("c")
    s0 = w * SPW

    pltpu.sync_copy(xt.at[pl.ds(s0, SPW)], idx_v)

    def _idx_ref(c):
        return idx_v.at[lax.div(c, B // CH), pl.ds(lax.rem(c, B // CH) * CH, CH)]

    def _out_ref(c):
        return out.at[s0 + lax.div(c, B // CH),
                      pl.ds(lax.rem(c, B // CH) * CH, CH)]

    for c in range(DEPTH):
        pltpu.async_copy(word.at[_idx_ref(c)], bufs[c], gsems[c])

    def _giter(g, _):
        for par in range(NBUF):
            c = g * NBUF + par
            buf = bufs[par]
            nxt = (par + DEPTH) % NBUF

            @pl.when(c + DEPTH < NCH)
            def _():
                @pl.when(c >= NBUF - DEPTH)
                def _():
                    pltpu.make_async_copy(bufs[nxt], _out_ref(0),
                                          ssems[nxt]).wait()

                pltpu.async_copy(word.at[_idx_ref(c + DEPTH)], bufs[nxt],
                                 gsems[nxt])

            pltpu.make_async_copy(word.at[pl.ds(0, CH)], buf,
                                  gsems[par]).wait()
            pltpu.async_copy(buf, _out_ref(c), ssems[par])
        return 0

    lax.fori_loop(0, NCH // NBUF, _giter, 0)
    for p in range(NBUF):
        pltpu.make_async_copy(bufs[p], _out_ref(0), ssems[p]).wait()


def _ln_body(scr, pos, typ, gamma, beta, out):
    e = scr[...] + pos[...][:, None, :] + typ[...][0][None, None, :]
    sum1 = jnp.sum(e, axis=-1, keepdims=True)
    sum2 = jnp.sum(e * e, axis=-1, keepdims=True)
    mean = sum1 * (1.0 / H)
    var = sum2 * (1.0 / H) - mean * mean
    a = lax.rsqrt(var + EPS)
    g = gamma[...][0] * SQRT_H
    b = beta[...][0] * SQRT_H
    out[...] = (e * a - mean * a) * g + b


_ln_kernel = pl.pallas_call(
    _ln_body,
    grid=(S // BS,),
    in_specs=[
        pl.BlockSpec((BS, B, H), lambda i: (i, 0, 0)),
        pl.BlockSpec((BS, H), lambda i: (i, 0)),
        pl.BlockSpec((2, H), lambda i: (0, 0)),
        pl.BlockSpec((1, H), lambda i: (0, 0)),
        pl.BlockSpec((1, H), lambda i: (0, 0)),
    ],
    out_specs=pl.BlockSpec((BS, B, H), lambda i: (i, 0, 0)),
    out_shape=jax.ShapeDtypeStruct((S, B, H), jnp.float32),
    compiler_params=pltpu.CompilerParams(
        dimension_semantics=("arbitrary",),
    ),
)


def kernel(x, word_emb, pos_emb, type_emb, ln_gamma, ln_beta):
    gathered = _gather_kernel(x.T, word_emb)
    return _ln_kernel(gathered, pos_emb, type_emb,
                      ln_gamma.reshape(1, H), ln_beta.reshape(1, H))
